# baseline (speedup 1.0000x reference)
import functools

import jax
import jax.numpy as jnp
from jax import lax
from jax.experimental import pallas as pl
from jax.experimental.pallas import tpu as pltpu
from jax.experimental.pallas import tpu_sc as plsc

_TM = 256
_NW = 32


def _matmul_body(te_ref, xs_ref, wgu_ref, wd_ref, ga_ref, gb_ref, ua_ref,
                 ub_ref, da_ref, db_ref, w_ref, scal_ref, out_ref, *, inter):
    s = scal_ref[0, 0]
    x = xs_ref[...].astype(jnp.bfloat16)
    gu = jnp.dot(x, wgu_ref[0], preferred_element_type=jnp.float32)

    xga = lax.dot_general(x, ga_ref[0], (((1,), (1,)), ((), ())),
                          preferred_element_type=jnp.float32)
    lg = lax.dot_general(xga, gb_ref[0], (((1,), (1,)), ((), ())),
                         preferred_element_type=jnp.float32)
    xua = lax.dot_general(x, ua_ref[0], (((1,), (1,)), ((), ())),
                          preferred_element_type=jnp.float32)
    lu = lax.dot_general(xua, ub_ref[0], (((1,), (1,)), ((), ())),
                         preferred_element_type=jnp.float32)

    gate = gu[:, :inter] + s * lg
    up = gu[:, inter:] + s * lu
    act = (jax.nn.silu(gate) * up).astype(jnp.bfloat16)

    y = jnp.dot(act, wd_ref[0], preferred_element_type=jnp.float32)
    ada = lax.dot_general(act, da_ref[0], (((1,), (1,)), ((), ())),
                          preferred_element_type=jnp.float32)
    ld = lax.dot_general(ada, db_ref[0], (((1,), (1,)), ((), ())),
                         preferred_element_type=jnp.float32)
    y = y + s * ld

    w = w_ref[0, 0, :]
    out_ref[...] = y * w[:, None]


def kernel(hidden_states, topk_ids, topk_weights, gate_a, gate_b, up_a, up_b,
           down_a, down_b, weight_indices, seq_lens, lora_ranks, scalings,
           base_gate_up_weight, base_down_weight):
    T, H = hidden_states.shape
    E, _, I2 = base_gate_up_weight.shape
    inter = I2 // 2
    R = gate_a.shape[2]
    K = topk_ids.shape[1]
    TK = T * K
    tm = _TM
    nt = TK // tm + E
    P = nt * tm

    e_flat = topk_ids.reshape(-1).astype(jnp.int32)
    w_flat = topk_weights.reshape(-1).astype(jnp.float32)
    order = jnp.argsort(e_flat).astype(jnp.int32)
    e_sorted = e_flat[order]
    tok_sorted = (order // K).astype(jnp.int32)
    w_sorted = w_flat[order]
    counts = jnp.bincount(e_flat, length=E).astype(jnp.int32)
    padded = ((counts + tm - 1) // tm) * tm
    cpad = jnp.cumsum(padded).astype(jnp.int32)
    pad_start = cpad - padded
    grp_start = (jnp.cumsum(counts) - counts).astype(jnp.int32)
    rank = jnp.arange(TK, dtype=jnp.int32) - grp_start[e_sorted]
    pos = pad_start[e_sorted] + rank
    tok_pad = jnp.zeros((P,), jnp.int32).at[pos].set(tok_sorted)
    w_pad = jnp.zeros((P,), jnp.float32).at[pos].set(w_sorted)
    pair_pos = jnp.zeros((TK,), jnp.int32).at[order].set(pos).reshape(T, K)
    tiles_end = cpad // tm
    tile_expert = jnp.minimum(
        jnp.searchsorted(tiles_end, jnp.arange(nt, dtype=jnp.int32),
                         side='right'),
        E - 1).astype(jnp.int32)

    adapter = weight_indices[0]
    ga = lax.dynamic_index_in_dim(gate_a, adapter, 0, False).astype(jnp.bfloat16)
    gb = lax.dynamic_index_in_dim(gate_b, adapter, 0, False).astype(jnp.bfloat16)
    ua = lax.dynamic_index_in_dim(up_a, adapter, 0, False).astype(jnp.bfloat16)
    ub = lax.dynamic_index_in_dim(up_b, adapter, 0, False).astype(jnp.bfloat16)
    da = lax.dynamic_index_in_dim(down_a, adapter, 0, False).astype(jnp.bfloat16)
    db = lax.dynamic_index_in_dim(down_b, adapter, 0, False).astype(jnp.bfloat16)
    scal = scalings[adapter].reshape(1, 1).astype(jnp.float32)

    x32 = hidden_states.astype(jnp.float32)
    mesh = plsc.VectorSubcoreMesh(core_axis_name="c", subcore_axis_name="s")

    rpw = P // _NW
    chunk = rpw // 2

    @functools.partial(
        pl.kernel, mesh=mesh,
        out_type=jax.ShapeDtypeStruct((P, H), jnp.float32),
        scratch_types=[pltpu.VMEM((chunk,), jnp.int32),
                       pltpu.VMEM((chunk, H), jnp.float32),
                       pltpu.SemaphoreType.DMA])
    def gather_rows(x_hbm, idx_hbm, out_hbm, idx_v, rows_v, sem):
        wid = lax.axis_index("s") * 2 + lax.axis_index("c")
        base = wid * rpw
        for c in range(rpw // chunk):
            off = base + c * chunk
            pltpu.sync_copy(idx_hbm.at[pl.ds(off, chunk)], idx_v)
            pltpu.async_copy(x_hbm.at[idx_v], rows_v, sem).wait()
            pltpu.sync_copy(rows_v, out_hbm.at[pl.ds(off, chunk)])

    xs = gather_rows(x32, tok_pad)

    grid_spec = pltpu.PrefetchScalarGridSpec(
        num_scalar_prefetch=1,
        grid=(nt,),
        in_specs=[
            pl.BlockSpec((tm, H), lambda g, te: (g, 0)),
            pl.BlockSpec((1, H, I2), lambda g, te: (te[g], 0, 0)),
            pl.BlockSpec((1, inter, H), lambda g, te: (te[g], 0, 0)),
            pl.BlockSpec((1, R, H), lambda g, te: (te[g], 0, 0)),
            pl.BlockSpec((1, inter, R), lambda g, te: (te[g], 0, 0)),
            pl.BlockSpec((1, R, H), lambda g, te: (te[g], 0, 0)),
            pl.BlockSpec((1, inter, R), lambda g, te: (te[g], 0, 0)),
            pl.BlockSpec((1, R, inter), lambda g, te: (te[g], 0, 0)),
            pl.BlockSpec((1, H, R), lambda g, te: (te[g], 0, 0)),
            pl.BlockSpec((1, 1, tm), lambda g, te: (g, 0, 0)),
            pl.BlockSpec(memory_space=pltpu.SMEM),
        ],
        out_specs=pl.BlockSpec((tm, H), lambda g, te: (g, 0)),
    )
    ys = pl.pallas_call(
        functools.partial(_matmul_body, inter=inter),
        grid_spec=grid_spec,
        out_shape=jax.ShapeDtypeStruct((P, H), jnp.float32),
    )(tile_expert, xs,
      base_gate_up_weight.astype(jnp.bfloat16),
      base_down_weight.astype(jnp.bfloat16),
      ga, gb, ua, ub, da, db,
      w_pad.reshape(nt, 1, tm), scal)

    ppw = (K * T) // _NW

    @functools.partial(
        pl.kernel, mesh=mesh,
        out_type=jax.ShapeDtypeStruct((K * T, H), jnp.float32),
        scratch_types=[pltpu.VMEM((ppw,), jnp.int32),
                       pltpu.VMEM((ppw, H), jnp.float32),
                       pltpu.SemaphoreType.DMA])
    def gather_pairs(ys_hbm, idx_hbm, out_hbm, idx_v, rows_v, sem):
        wid = lax.axis_index("s") * 2 + lax.axis_index("c")
        base = wid * ppw
        pltpu.sync_copy(idx_hbm.at[pl.ds(base, ppw)], idx_v)
        pltpu.async_copy(ys_hbm.at[idx_v], rows_v, sem).wait()
        pltpu.sync_copy(rows_v, out_hbm.at[pl.ds(base, ppw)])

    idx_all = jnp.concatenate([pair_pos[:, k] for k in range(K)])
    pairs = gather_pairs(ys, idx_all)

    tm2 = 256

    def _add_body(*refs):
        out_ref = refs[-1]
        acc = refs[0][...]
        for r in refs[1:-1]:
            acc = acc + r[...]
        out_ref[...] = acc

    out = pl.pallas_call(
        _add_body,
        grid=(T // tm2,),
        in_specs=[pl.BlockSpec((tm2, H), lambda t: (t, 0))
                  for _ in range(K)],
        out_specs=pl.BlockSpec((tm2, H), lambda t: (t, 0)),
        out_shape=jax.ShapeDtypeStruct((T, H), jnp.float32),
    )(*[lax.slice_in_dim(pairs, k * T, (k + 1) * T, axis=0)
        for k in range(K)])
    return out.astype(hidden_states.dtype)


# speedup vs baseline: 1.3110x; 1.3110x over previous
import functools

import jax
import jax.numpy as jnp
from jax import lax
from jax.experimental import pallas as pl
from jax.experimental.pallas import tpu as pltpu
from jax.experimental.pallas import tpu_sc as plsc

_TM = 256
_NW = 32


def _matmul_body(te_ref, xs_ref, wgu_ref, wd_ref, ga_ref, gb_ref, ua_ref,
                 ub_ref, da_ref, db_ref, w_ref, scal_ref, out_ref, *, inter):
    s = scal_ref[0, 0]
    x = xs_ref[...].astype(jnp.bfloat16)
    gu = jnp.dot(x, wgu_ref[0], preferred_element_type=jnp.float32)

    xga = lax.dot_general(x, ga_ref[0], (((1,), (1,)), ((), ())),
                          preferred_element_type=jnp.float32)
    lg = lax.dot_general(xga, gb_ref[0], (((1,), (1,)), ((), ())),
                         preferred_element_type=jnp.float32)
    xua = lax.dot_general(x, ua_ref[0], (((1,), (1,)), ((), ())),
                          preferred_element_type=jnp.float32)
    lu = lax.dot_general(xua, ub_ref[0], (((1,), (1,)), ((), ())),
                         preferred_element_type=jnp.float32)

    gate = gu[:, :inter] + s * lg
    up = gu[:, inter:] + s * lu
    act = (jax.nn.silu(gate) * up).astype(jnp.bfloat16)

    y = jnp.dot(act, wd_ref[0], preferred_element_type=jnp.float32)
    ada = lax.dot_general(act, da_ref[0], (((1,), (1,)), ((), ())),
                          preferred_element_type=jnp.float32)
    ld = lax.dot_general(ada, db_ref[0], (((1,), (1,)), ((), ())),
                         preferred_element_type=jnp.float32)
    y = y + s * ld

    w = w_ref[0, 0, :]
    out_ref[...] = y * w[:, None]


def kernel(hidden_states, topk_ids, topk_weights, gate_a, gate_b, up_a, up_b,
           down_a, down_b, weight_indices, seq_lens, lora_ranks, scalings,
           base_gate_up_weight, base_down_weight):
    T, H = hidden_states.shape
    E, _, I2 = base_gate_up_weight.shape
    inter = I2 // 2
    R = gate_a.shape[2]
    K = topk_ids.shape[1]
    TK = T * K
    tm = _TM
    nt = TK // tm + E
    P = nt * tm

    e_flat = topk_ids.reshape(-1).astype(jnp.int32)
    w_flat = topk_weights.reshape(-1).astype(jnp.float32)
    order = jnp.argsort(e_flat).astype(jnp.int32)
    e_sorted = e_flat[order]
    tok_sorted = (order // K).astype(jnp.int32)
    w_sorted = w_flat[order]
    counts = jnp.bincount(e_flat, length=E).astype(jnp.int32)
    padded = ((counts + tm - 1) // tm) * tm
    cpad = jnp.cumsum(padded).astype(jnp.int32)
    pad_start = cpad - padded
    grp_start = (jnp.cumsum(counts) - counts).astype(jnp.int32)
    rank = jnp.arange(TK, dtype=jnp.int32) - grp_start[e_sorted]
    pos = pad_start[e_sorted] + rank
    tok_pad = (jnp.arange(P, dtype=jnp.int32) % T).at[pos].set(tok_sorted)
    w_pad = jnp.zeros((P,), jnp.float32).at[pos].set(w_sorted)
    pair_pos = jnp.zeros((TK,), jnp.int32).at[order].set(pos).reshape(T, K)
    tiles_end = cpad // tm
    tile_expert = jnp.minimum(
        jnp.searchsorted(tiles_end, jnp.arange(nt, dtype=jnp.int32),
                         side='right'),
        E - 1).astype(jnp.int32)

    adapter = weight_indices[0]
    ga = lax.dynamic_index_in_dim(gate_a, adapter, 0, False).astype(jnp.bfloat16)
    gb = lax.dynamic_index_in_dim(gate_b, adapter, 0, False).astype(jnp.bfloat16)
    ua = lax.dynamic_index_in_dim(up_a, adapter, 0, False).astype(jnp.bfloat16)
    ub = lax.dynamic_index_in_dim(up_b, adapter, 0, False).astype(jnp.bfloat16)
    da = lax.dynamic_index_in_dim(down_a, adapter, 0, False).astype(jnp.bfloat16)
    db = lax.dynamic_index_in_dim(down_b, adapter, 0, False).astype(jnp.bfloat16)
    scal = scalings[adapter].reshape(1, 1).astype(jnp.float32)

    x32 = hidden_states.astype(jnp.float32)
    mesh = plsc.VectorSubcoreMesh(core_axis_name="c", subcore_axis_name="s")

    rpw = P // _NW
    chunk = rpw // 2

    @functools.partial(
        pl.kernel, mesh=mesh,
        out_type=jax.ShapeDtypeStruct((P, H), jnp.float32),
        scratch_types=[pltpu.VMEM((chunk,), jnp.int32),
                       pltpu.VMEM((chunk, H), jnp.float32),
                       pltpu.SemaphoreType.DMA])
    def gather_rows(x_hbm, idx_hbm, out_hbm, idx_v, rows_v, sem):
        wid = lax.axis_index("s") * 2 + lax.axis_index("c")
        base = wid * rpw
        for c in range(rpw // chunk):
            off = base + c * chunk
            pltpu.sync_copy(idx_hbm.at[pl.ds(off, chunk)], idx_v)
            pltpu.async_copy(x_hbm.at[idx_v], rows_v, sem).wait()
            pltpu.sync_copy(rows_v, out_hbm.at[pl.ds(off, chunk)])

    xs = gather_rows(x32, tok_pad)

    grid_spec = pltpu.PrefetchScalarGridSpec(
        num_scalar_prefetch=1,
        grid=(nt,),
        in_specs=[
            pl.BlockSpec((tm, H), lambda g, te: (g, 0)),
            pl.BlockSpec((1, H, I2), lambda g, te: (te[g], 0, 0)),
            pl.BlockSpec((1, inter, H), lambda g, te: (te[g], 0, 0)),
            pl.BlockSpec((1, R, H), lambda g, te: (te[g], 0, 0)),
            pl.BlockSpec((1, inter, R), lambda g, te: (te[g], 0, 0)),
            pl.BlockSpec((1, R, H), lambda g, te: (te[g], 0, 0)),
            pl.BlockSpec((1, inter, R), lambda g, te: (te[g], 0, 0)),
            pl.BlockSpec((1, R, inter), lambda g, te: (te[g], 0, 0)),
            pl.BlockSpec((1, H, R), lambda g, te: (te[g], 0, 0)),
            pl.BlockSpec((1, 1, tm), lambda g, te: (g, 0, 0)),
            pl.BlockSpec(memory_space=pltpu.SMEM),
        ],
        out_specs=pl.BlockSpec((tm, H), lambda g, te: (g, 0)),
    )
    ys = pl.pallas_call(
        functools.partial(_matmul_body, inter=inter),
        grid_spec=grid_spec,
        out_shape=jax.ShapeDtypeStruct((P, H), jnp.float32),
    )(tile_expert, xs,
      base_gate_up_weight.astype(jnp.bfloat16),
      base_down_weight.astype(jnp.bfloat16),
      ga, gb, ua, ub, da, db,
      w_pad.reshape(nt, 1, tm), scal)

    ppw = (K * T) // _NW

    @functools.partial(
        pl.kernel, mesh=mesh,
        out_type=jax.ShapeDtypeStruct((K * T, H), jnp.float32),
        scratch_types=[pltpu.VMEM((ppw,), jnp.int32),
                       pltpu.VMEM((ppw, H), jnp.float32),
                       pltpu.SemaphoreType.DMA])
    def gather_pairs(ys_hbm, idx_hbm, out_hbm, idx_v, rows_v, sem):
        wid = lax.axis_index("s") * 2 + lax.axis_index("c")
        base = wid * ppw
        pltpu.sync_copy(idx_hbm.at[pl.ds(base, ppw)], idx_v)
        pltpu.async_copy(ys_hbm.at[idx_v], rows_v, sem).wait()
        pltpu.sync_copy(rows_v, out_hbm.at[pl.ds(base, ppw)])

    idx_all = jnp.concatenate([pair_pos[:, k] for k in range(K)])
    pairs = gather_pairs(ys, idx_all)

    tm2 = 256

    def _add_body(*refs):
        out_ref = refs[-1]
        acc = refs[0][...]
        for r in refs[1:-1]:
            acc = acc + r[...]
        out_ref[...] = acc

    out = pl.pallas_call(
        _add_body,
        grid=(T // tm2,),
        in_specs=[pl.BlockSpec((tm2, H), lambda t: (t, 0))
                  for _ in range(K)],
        out_specs=pl.BlockSpec((tm2, H), lambda t: (t, 0)),
        out_shape=jax.ShapeDtypeStruct((T, H), jnp.float32),
    )(*[lax.slice_in_dim(pairs, k * T, (k + 1) * T, axis=0)
        for k in range(K)])
    return out.astype(hidden_states.dtype)


# speedup vs baseline: 1.5032x; 1.1466x over previous
import functools

import jax
import jax.numpy as jnp
from jax import lax
from jax.experimental import pallas as pl
from jax.experimental.pallas import tpu as pltpu
from jax.experimental.pallas import tpu_sc as plsc

_TM = 256
_NW = 32


def _matmul_body(te_ref, xs_ref, wgu_ref, wd_ref, ga_ref, gb_ref, ua_ref,
                 ub_ref, da_ref, db_ref, w_ref, scal_ref, out_ref, *, inter):
    s = scal_ref[0, 0]
    x = xs_ref[...]
    gu = jnp.dot(x, wgu_ref[0], preferred_element_type=jnp.float32)

    xga = lax.dot_general(x, ga_ref[0], (((1,), (1,)), ((), ())),
                          preferred_element_type=jnp.float32)
    lg = lax.dot_general(xga, gb_ref[0], (((1,), (1,)), ((), ())),
                         preferred_element_type=jnp.float32)
    xua = lax.dot_general(x, ua_ref[0], (((1,), (1,)), ((), ())),
                          preferred_element_type=jnp.float32)
    lu = lax.dot_general(xua, ub_ref[0], (((1,), (1,)), ((), ())),
                         preferred_element_type=jnp.float32)

    gate = gu[:, :inter] + s * lg
    up = gu[:, inter:] + s * lu
    act = jax.nn.silu(gate) * up

    y = jnp.dot(act, wd_ref[0], preferred_element_type=jnp.float32)
    ada = lax.dot_general(act, da_ref[0], (((1,), (1,)), ((), ())),
                          preferred_element_type=jnp.float32)
    ld = lax.dot_general(ada, db_ref[0], (((1,), (1,)), ((), ())),
                         preferred_element_type=jnp.float32)
    y = y + s * ld

    w = w_ref[0, 0, :]
    out_ref[...] = y * w[:, None]


def kernel(hidden_states, topk_ids, topk_weights, gate_a, gate_b, up_a, up_b,
           down_a, down_b, weight_indices, seq_lens, lora_ranks, scalings,
           base_gate_up_weight, base_down_weight):
    T, H = hidden_states.shape
    E, _, I2 = base_gate_up_weight.shape
    inter = I2 // 2
    R = gate_a.shape[2]
    K = topk_ids.shape[1]
    TK = T * K
    tm = _TM
    nt = TK // tm + E
    P = nt * tm

    e_flat = topk_ids.reshape(-1).astype(jnp.int32)
    w_flat = topk_weights.reshape(-1).astype(jnp.float32)
    order = jnp.argsort(e_flat).astype(jnp.int32)
    e_sorted = e_flat[order]
    tok_sorted = (order // K).astype(jnp.int32)
    w_sorted = w_flat[order]
    counts = jnp.bincount(e_flat, length=E).astype(jnp.int32)
    padded = ((counts + tm - 1) // tm) * tm
    cpad = jnp.cumsum(padded).astype(jnp.int32)
    pad_start = cpad - padded
    grp_start = (jnp.cumsum(counts) - counts).astype(jnp.int32)
    rank = jnp.arange(TK, dtype=jnp.int32) - grp_start[e_sorted]
    pos = pad_start[e_sorted] + rank
    tok_pad = (jnp.arange(P, dtype=jnp.int32) % T).at[pos].set(tok_sorted)
    w_pad = jnp.zeros((P,), jnp.float32).at[pos].set(w_sorted)
    pair_pos = jnp.zeros((TK,), jnp.int32).at[order].set(pos).reshape(T, K)
    tiles_end = cpad // tm
    tile_expert = jnp.minimum(
        jnp.searchsorted(tiles_end, jnp.arange(nt, dtype=jnp.int32),
                         side='right'),
        E - 1).astype(jnp.int32)

    adapter = weight_indices[0]
    ga = lax.dynamic_index_in_dim(gate_a, adapter, 0, False)
    gb = lax.dynamic_index_in_dim(gate_b, adapter, 0, False)
    ua = lax.dynamic_index_in_dim(up_a, adapter, 0, False)
    ub = lax.dynamic_index_in_dim(up_b, adapter, 0, False)
    da = lax.dynamic_index_in_dim(down_a, adapter, 0, False)
    db = lax.dynamic_index_in_dim(down_b, adapter, 0, False)
    scal = scalings[adapter].reshape(1, 1).astype(jnp.float32)

    x32 = hidden_states.astype(jnp.float32)
    mesh = plsc.VectorSubcoreMesh(core_axis_name="c", subcore_axis_name="s")

    rpw = P // _NW
    chunk = rpw // 2

    @functools.partial(
        pl.kernel, mesh=mesh,
        out_type=jax.ShapeDtypeStruct((P, H), jnp.float32),
        scratch_types=[pltpu.VMEM((chunk,), jnp.int32),
                       pltpu.VMEM((chunk, H), jnp.float32),
                       pltpu.SemaphoreType.DMA])
    def gather_rows(x_hbm, idx_hbm, out_hbm, idx_v, rows_v, sem):
        wid = lax.axis_index("s") * 2 + lax.axis_index("c")
        base = wid * rpw
        for c in range(rpw // chunk):
            off = base + c * chunk
            pltpu.sync_copy(idx_hbm.at[pl.ds(off, chunk)], idx_v)
            pltpu.async_copy(x_hbm.at[idx_v], rows_v, sem).wait()
            pltpu.sync_copy(rows_v, out_hbm.at[pl.ds(off, chunk)])

    xs = gather_rows(x32, tok_pad)

    grid_spec = pltpu.PrefetchScalarGridSpec(
        num_scalar_prefetch=1,
        grid=(nt,),
        in_specs=[
            pl.BlockSpec((tm, H), lambda g, te: (g, 0)),
            pl.BlockSpec((1, H, I2), lambda g, te: (te[g], 0, 0)),
            pl.BlockSpec((1, inter, H), lambda g, te: (te[g], 0, 0)),
            pl.BlockSpec((1, R, H), lambda g, te: (te[g], 0, 0)),
            pl.BlockSpec((1, inter, R), lambda g, te: (te[g], 0, 0)),
            pl.BlockSpec((1, R, H), lambda g, te: (te[g], 0, 0)),
            pl.BlockSpec((1, inter, R), lambda g, te: (te[g], 0, 0)),
            pl.BlockSpec((1, R, inter), lambda g, te: (te[g], 0, 0)),
            pl.BlockSpec((1, H, R), lambda g, te: (te[g], 0, 0)),
            pl.BlockSpec((1, 1, tm), lambda g, te: (g, 0, 0)),
            pl.BlockSpec(memory_space=pltpu.SMEM),
        ],
        out_specs=pl.BlockSpec((tm, H), lambda g, te: (g, 0)),
    )
    ys = pl.pallas_call(
        functools.partial(_matmul_body, inter=inter),
        grid_spec=grid_spec,
        out_shape=jax.ShapeDtypeStruct((P, H), jnp.float32),
    )(tile_expert, xs, base_gate_up_weight, base_down_weight,
      ga, gb, ua, ub, da, db,
      w_pad.reshape(nt, 1, tm), scal)

    ppw = (K * T) // _NW

    @functools.partial(
        pl.kernel, mesh=mesh,
        out_type=jax.ShapeDtypeStruct((K * T, H), jnp.float32),
        scratch_types=[pltpu.VMEM((ppw,), jnp.int32),
                       pltpu.VMEM((ppw, H), jnp.float32),
                       pltpu.SemaphoreType.DMA])
    def gather_pairs(ys_hbm, idx_hbm, out_hbm, idx_v, rows_v, sem):
        wid = lax.axis_index("s") * 2 + lax.axis_index("c")
        base = wid * ppw
        pltpu.sync_copy(idx_hbm.at[pl.ds(base, ppw)], idx_v)
        pltpu.async_copy(ys_hbm.at[idx_v], rows_v, sem).wait()
        pltpu.sync_copy(rows_v, out_hbm.at[pl.ds(base, ppw)])

    idx_all = jnp.concatenate([pair_pos[:, k] for k in range(K)])
    pairs = gather_pairs(ys, idx_all)

    tm2 = 256

    def _add_body(*refs):
        out_ref = refs[-1]
        acc = refs[0][...]
        for r in refs[1:-1]:
            acc = acc + r[...]
        out_ref[...] = acc

    out = pl.pallas_call(
        _add_body,
        grid=(T // tm2,),
        in_specs=[pl.BlockSpec((tm2, H), lambda t: (t, 0))
                  for _ in range(K)],
        out_specs=pl.BlockSpec((tm2, H), lambda t: (t, 0)),
        out_shape=jax.ShapeDtypeStruct((T, H), jnp.float32),
    )(*[lax.slice_in_dim(pairs, k * T, (k + 1) * T, axis=0)
        for k in range(K)])
    return out.astype(hidden_states.dtype)
